# baseline (speedup 1.0000x reference)
import numpy as np
import jax
import jax.numpy as jnp
from jax.experimental import pallas as pl
from jax.experimental.pallas import tpu as pltpu

DIM = 384
NHEADS = 6
HDIM = DIM // NHEADS
NANCH = 8
A = NANCH * NANCH
PDIM = 2 * DIM

_R = np.arange(DIM)[:, None]
_C = np.arange(DIM)[None, :]
_BDMASK = ((_R // HDIM) == (_C // HDIM)).astype(np.float32)
_BD_LOWER = (_BDMASK * (_R > _C)).astype(np.float32)
_R3 = np.arange(3 * DIM)[:, None] % DIM
_BD3 = ((_R3 // HDIM) == (_C // HDIM)).astype(np.float32)
_ONESCOL = (np.arange(2 * HDIM)[None, :] == 0) * np.ones((1024, 1))


def _cluster_kernel(x_ref, pw_ref, pb_ref, ow_ref, ob_ref, al_ref, be_ref,
                    bdm_ref, bdl_ref, bd3_ref, oc_ref, out_ref, pwp_ref, pbp_ref):
    f32 = jnp.float32
    bf16 = jnp.bfloat16
    hw = x_ref.shape[1]
    dd = (((1,), (0,)), ((), ()))

    def mm(a, b, dims=((1,), (0,))):
        return jax.lax.dot_general(a, b, (dims, ((), ())),
                                   preferred_element_type=f32)

    @pl.when(pl.program_id(0) == 0)
    def _():
        pw = pw_ref[...]
        pwp_ref[...] = jnp.concatenate(
            [pw[2 * h * HDIM:(2 * h + 1) * HDIM] for h in range(NHEADS)]
            + [pw[(2 * h + 1) * HDIM:(2 * h + 2) * HDIM]
               for h in range(NHEADS)],
            axis=0).astype(bf16)
        pb = pb_ref[...]
        pbp_ref[...] = jnp.concatenate(
            [pb[:, 2 * h * HDIM:(2 * h + 1) * HDIM] for h in range(NHEADS)]
            + [pb[:, (2 * h + 1) * HDIM:(2 * h + 2) * HDIM]
               for h in range(NHEADS)],
            axis=1)

    bdmask = bdm_ref[...] > jnp.bfloat16(0.5)

    xb = x_ref[0]
    xp = mm(xb.astype(bf16), pwp_ref[...], ((1,), (1,))) + pbp_ref[...]

    x3 = xp.reshape(hw // 4, 4, PDIM)
    m1 = jnp.maximum(jnp.maximum(x3[:, 0], x3[:, 1]),
                     jnp.maximum(x3[:, 2], x3[:, 3]))
    m4 = m1.reshape(NANCH, 4, NANCH, PDIM)
    m2 = jnp.maximum(jnp.maximum(m4[:, 0], m4[:, 1]),
                     jnp.maximum(m4[:, 2], m4[:, 3]))
    pooled = m2.reshape(A, PDIM)

    P0, V0 = xp[:, :DIM], xp[:, DIM:]
    P1, V1 = pooled[:, :DIM], pooled[:, DIM:]

    sq = P0 * P0
    hi = sq.astype(bf16)
    r1 = sq - hi.astype(f32)
    mid = r1.astype(bf16)
    lo = (r1 - mid.astype(f32)).astype(bf16)
    cat = jnp.concatenate([hi, mid, lo], axis=1)
    ss = jax.lax.dot_general(cat, bd3_ref[...], dd,
                             preferred_element_type=f32)
    P0n = P0 * jax.lax.rsqrt(jnp.maximum(ss, 1e-24))

    p1n = []
    for h in range(NHEADS):
        p1h = P1[:, h * HDIM:(h + 1) * HDIM]
        n1 = jnp.sqrt(jnp.sum(p1h * p1h, axis=1, keepdims=True))
        p1n.append(p1h / jnp.maximum(n1, 1e-12))
    P1n = jnp.concatenate(p1n, axis=1)

    P1BD = jnp.where(bdmask, jnp.concatenate([P1n.T] * NHEADS, axis=1),
                     0.0).astype(bf16)
    sim = mm(P0n.astype(bf16), P1BD)
    sim = jax.nn.sigmoid(al_ref[0, 0] * sim + be_ref[0, 0])

    eqs = []
    for h in range(NHEADS):
        sh = sim[:, h * A:(h + 1) * A]
        mx = jnp.max(sh, axis=1, keepdims=True)
        eqs.append(jnp.where(sh >= mx, 1.0, 0.0).astype(bf16))
    eqf = jnp.concatenate(eqs, axis=1)
    nearlier = mm(eqf, bdl_ref[...])
    half = jnp.bfloat16(0.5)
    Sb = jnp.where((eqf > half) & (nearlier < 0.5), sim.astype(bf16),
                   jnp.bfloat16(0.0))

    V0e = jnp.concatenate([V0.astype(bf16), oc_ref[...]], axis=1)
    M1e = mm(Sb, V0e, ((0,), (0,)))
    denomc = M1e[:, DIM:DIM + 1] + 1.0
    MSG = jnp.where(bdmask,
                    M1e[:, :DIM] + jnp.concatenate([V1] * NHEADS, axis=0),
                    0.0)
    MSG = MSG / denomc

    W2 = mm(MSG.astype(bf16), ow_ref[...].astype(bf16), ((1,), (1,)))
    y = mm(Sb, W2.astype(bf16))
    out_ref[0] = y + ob_ref[...]


def kernel(x, proj_w, proj_b, out_w, out_b, alpha, beta):
    n, h, w, _ = x.shape
    hw = h * w
    full = lambda i: (0, 0)
    out = pl.pallas_call(
        _cluster_kernel,
        grid=(n,),
        in_specs=[
            pl.BlockSpec((1, hw, DIM), lambda i: (i, 0, 0)),
            pl.BlockSpec((PDIM, DIM), full),
            pl.BlockSpec((1, PDIM), full),
            pl.BlockSpec((DIM, DIM), full),
            pl.BlockSpec((1, DIM), full),
            pl.BlockSpec((1, 1), full),
            pl.BlockSpec((1, 1), full),
            pl.BlockSpec((DIM, DIM), full),
            pl.BlockSpec((DIM, DIM), full),
            pl.BlockSpec((3 * DIM, DIM), full),
            pl.BlockSpec((1024, 2 * HDIM), full),
        ],
        out_specs=pl.BlockSpec((1, hw, DIM), lambda i: (i, 0, 0)),
        out_shape=jax.ShapeDtypeStruct((n, hw, DIM), jnp.float32),
        scratch_shapes=[
            pltpu.VMEM((PDIM, DIM), jnp.bfloat16),
            pltpu.VMEM((1, PDIM), jnp.float32),
        ],
    )(x.reshape(n, hw, DIM), proj_w, proj_b.reshape(1, PDIM), out_w,
      out_b.reshape(1, DIM), alpha.reshape(1, 1), beta.reshape(1, 1),
      jnp.asarray(_BDMASK, dtype=jnp.bfloat16),
      jnp.asarray(_BD_LOWER, dtype=jnp.bfloat16),
      jnp.asarray(_BD3, dtype=jnp.bfloat16),
      jnp.asarray(_ONESCOL, dtype=jnp.bfloat16))
    return out.reshape(n, h, w, DIM)
